# baseline (speedup 1.0000x reference)
import jax
import jax.numpy as jnp
from jax import lax
from jax.experimental import pallas as pl
from jax.experimental.pallas import tpu as pltpu
from jax.experimental.pallas import tpu_sc as plsc

NUM_ENT = 100000
NUM_REL = 1000
ENT_DIM = 90
TIME_DIM = 10
BATCH = 16384
REL_DIM = ENT_DIM + TIME_DIM

_NC, _NS = 2, 16
_NW = _NC * _NS
_BPW = BATCH // _NW
_CHUNK = 128
_NCHUNK = _BPW // _CHUNK


def _sc_gather_body(heads_hbm, tails_hbm, rels_hbm, ent_hbm, relemb_hbm,
                    yf, yp, ya, mf, mp, ma, df, dp, da,
                    h_out, t_out, r_out, ht_out, tt_out,
                    hidx, tidx, ridx, hrows, trows, rrows, htbuf, ttbuf, sem):
    wid = lax.axis_index("s") * _NC + lax.axis_index("c")
    tabs = (yf, yp, ya, mf, mp, ma, df, dp, da)
    for ci in range(_NCHUNK):
        base = wid * _BPW + ci * _CHUNK
        pltpu.sync_copy(heads_hbm.at[pl.ds(base, _CHUNK)], hidx)
        pltpu.sync_copy(tails_hbm.at[pl.ds(base, _CHUNK)], tidx)
        pltpu.sync_copy(rels_hbm.at[pl.ds(base, _CHUNK)], ridx)
        descs = [
            pltpu.async_copy(ent_hbm.at[hidx], hrows, sem),
            pltpu.async_copy(ent_hbm.at[tidx], trows, sem),
            pltpu.async_copy(relemb_hbm.at[ridx], rrows, sem),
        ]
        for k in range(9):
            descs.append(pltpu.async_copy(tabs[k].at[hidx], htbuf.at[k], sem))
            descs.append(pltpu.async_copy(tabs[k].at[tidx], ttbuf.at[k], sem))
        for d in descs:
            d.wait()
        pltpu.sync_copy(hrows, h_out.at[pl.ds(base, _CHUNK)])
        pltpu.sync_copy(trows, t_out.at[pl.ds(base, _CHUNK)])
        pltpu.sync_copy(rrows, r_out.at[pl.ds(base, _CHUNK)])
        for k in range(9):
            pltpu.sync_copy(htbuf.at[k], ht_out.at[k, pl.ds(base, _CHUNK)])
            pltpu.sync_copy(ttbuf.at[k], tt_out.at[k, pl.ds(base, _CHUNK)])


_sc_gather_cache = []


def _get_sc_gather():
    if not _sc_gather_cache:
        _sc_gather_cache.append(_build_sc_gather())
    return _sc_gather_cache[0]


def _build_sc_gather():
    return pl.kernel(
        _sc_gather_body,
        out_type=(
            jax.ShapeDtypeStruct((BATCH, ENT_DIM), jnp.float32),
            jax.ShapeDtypeStruct((BATCH, ENT_DIM), jnp.float32),
            jax.ShapeDtypeStruct((BATCH, REL_DIM), jnp.float32),
            jax.ShapeDtypeStruct((9, BATCH, TIME_DIM), jnp.float32),
            jax.ShapeDtypeStruct((9, BATCH, TIME_DIM), jnp.float32),
        ),
        mesh=plsc.VectorSubcoreMesh(core_axis_name="c", subcore_axis_name="s",
                                    num_cores=_NC, num_subcores=_NS),
        compiler_params=pltpu.CompilerParams(use_tc_tiling_on_sc=False),
        scratch_types=[
            pltpu.VMEM((_CHUNK,), jnp.int32),
            pltpu.VMEM((_CHUNK,), jnp.int32),
            pltpu.VMEM((_CHUNK,), jnp.int32),
            pltpu.VMEM((_CHUNK, ENT_DIM), jnp.float32),
            pltpu.VMEM((_CHUNK, ENT_DIM), jnp.float32),
            pltpu.VMEM((_CHUNK, REL_DIM), jnp.float32),
            pltpu.VMEM((9, _CHUNK, TIME_DIM), jnp.float32),
            pltpu.VMEM((9, _CHUNK, TIME_DIM), jnp.float32),
            pltpu.SemaphoreType.DMA,
        ],
    )

_BLK = 512


def _tc_score_body(h_ref, t_ref, r_ref, ht_ref, tt_ref, y_ref, mo_ref, dy_ref,
                   out_ref):
    y = y_ref[...]
    mo = mo_ref[...]
    dy = dy_ref[...]

    def tenc(b):
        return (b[2] * jnp.sin(b[0] * y + b[1])
                + b[5] * jnp.sin(b[3] * mo + b[4])
                + b[8] * jnp.sin(b[6] * dy + b[7]))

    h = h_ref[...]
    t = t_ref[...]
    r = r_ref[...]
    d90 = h + r[:, :ENT_DIM] - t
    d10 = tenc(ht_ref[...]) + r[:, ENT_DIM:] - tenc(tt_ref[...])
    s = jnp.sum(d90 * d90, axis=1) + jnp.sum(d10 * d10, axis=1)
    out_ref[...] = jnp.sqrt(s)[:, None]


def _tc_score(h, t, r, ht, tt, y2, m2, d2, interpret=False):
    return pl.pallas_call(
        _tc_score_body,
        grid=(BATCH // _BLK,),
        in_specs=[
            pl.BlockSpec((_BLK, ENT_DIM), lambda i: (i, 0)),
            pl.BlockSpec((_BLK, ENT_DIM), lambda i: (i, 0)),
            pl.BlockSpec((_BLK, REL_DIM), lambda i: (i, 0)),
            pl.BlockSpec((9, _BLK, TIME_DIM), lambda i: (0, i, 0)),
            pl.BlockSpec((9, _BLK, TIME_DIM), lambda i: (0, i, 0)),
            pl.BlockSpec((_BLK, 1), lambda i: (i, 0)),
            pl.BlockSpec((_BLK, 1), lambda i: (i, 0)),
            pl.BlockSpec((_BLK, 1), lambda i: (i, 0)),
        ],
        out_specs=pl.BlockSpec((_BLK, 1), lambda i: (i, 0)),
        out_shape=jax.ShapeDtypeStruct((BATCH, 1), jnp.float32),
        interpret=interpret,
    )(h, t, r, ht, tt, y2, m2, d2)


def kernel(heads, rels, tails, years, months, days, entity_emb, relation_emb,
           year_freq, month_freq, day_freq, year_phi, month_phi, day_phi,
           year_amp, month_amp, day_amp):
    hi = heads.astype(jnp.int32)
    ti = tails.astype(jnp.int32)
    ri = rels.astype(jnp.int32)
    h, t, r, ht, tt = _get_sc_gather()(hi, ti, ri, entity_emb, relation_emb,
                                 year_freq, year_phi, year_amp,
                                 month_freq, month_phi, month_amp,
                                 day_freq, day_phi, day_amp)
    y2 = years.reshape(BATCH, 1)
    m2 = months.reshape(BATCH, 1)
    d2 = days.reshape(BATCH, 1)
    scores = _tc_score(h, t, r, ht, tt, y2, m2, d2)
    return scores.reshape(-1)


# speedup vs baseline: 1.4402x; 1.4402x over previous
import functools
import jax
import jax.numpy as jnp
from jax import lax
from jax.experimental import pallas as pl
from jax.experimental.pallas import tpu as pltpu
from jax.experimental.pallas import tpu_sc as plsc

NUM_ENT = 100000
NUM_REL = 1000
ENT_DIM = 90
TIME_DIM = 10
BATCH = 16384
MEGA_W = 256
REL_W = 128

_NC, _NS = 2, 16
_NW = _NC * _NS
_BPW = BATCH // _NW
_CHUNK = 128
_NCHUNK = _BPW // _CHUNK


def _sc_gather_body(heads_hbm, tails_hbm, rels_hbm, mega_hbm, rel_hbm,
                    h_out, t_out, r_out,
                    hidx, tidx, ridx, hrows, trows, rrows, sem):
    wid = lax.axis_index("s") * _NC + lax.axis_index("c")
    for ci in range(_NCHUNK):
        base = wid * _BPW + ci * _CHUNK
        pltpu.sync_copy(heads_hbm.at[pl.ds(base, _CHUNK)], hidx)
        pltpu.sync_copy(tails_hbm.at[pl.ds(base, _CHUNK)], tidx)
        pltpu.sync_copy(rels_hbm.at[pl.ds(base, _CHUNK)], ridx)
        d1 = pltpu.async_copy(mega_hbm.at[hidx], hrows, sem)
        d2 = pltpu.async_copy(mega_hbm.at[tidx], trows, sem)
        d3 = pltpu.async_copy(rel_hbm.at[ridx], rrows, sem)
        d1.wait()
        d2.wait()
        d3.wait()
        pltpu.sync_copy(hrows, h_out.at[pl.ds(base, _CHUNK)])
        pltpu.sync_copy(trows, t_out.at[pl.ds(base, _CHUNK)])
        pltpu.sync_copy(rrows, r_out.at[pl.ds(base, _CHUNK)])


_sc_gather_cache = []


def _get_sc_gather():
    if not _sc_gather_cache:
        _sc_gather_cache.append(_build_sc_gather())
    return _sc_gather_cache[0]


def _build_sc_gather():
    return pl.kernel(
        _sc_gather_body,
        out_type=(
            jax.ShapeDtypeStruct((BATCH, MEGA_W), jnp.float32),
            jax.ShapeDtypeStruct((BATCH, MEGA_W), jnp.float32),
            jax.ShapeDtypeStruct((BATCH, REL_W), jnp.float32),
        ),
        mesh=plsc.VectorSubcoreMesh(core_axis_name="c", subcore_axis_name="s",
                                    num_cores=_NC, num_subcores=_NS),
        compiler_params=pltpu.CompilerParams(use_tc_tiling_on_sc=True),
        scratch_types=[
            pltpu.VMEM((_CHUNK,), jnp.int32),
            pltpu.VMEM((_CHUNK,), jnp.int32),
            pltpu.VMEM((_CHUNK,), jnp.int32),
            pltpu.VMEM((_CHUNK, MEGA_W), jnp.float32),
            pltpu.VMEM((_CHUNK, MEGA_W), jnp.float32),
            pltpu.VMEM((_CHUNK, REL_W), jnp.float32),
            pltpu.SemaphoreType.DMA,
        ],
    )


_BLK = 512


def _tc_score_body(h_ref, t_ref, r_ref, y_ref, mo_ref, dy_ref, out_ref):
    y = y_ref[...]
    mo = mo_ref[...]
    dy = dy_ref[...]
    tvec = jnp.concatenate(
        [jnp.broadcast_to(y, (_BLK, TIME_DIM)),
         jnp.broadcast_to(mo, (_BLK, TIME_DIM)),
         jnp.broadcast_to(dy, (_BLK, TIME_DIM)),
         jnp.zeros((_BLK, 2), jnp.float32)], axis=1)

    h = h_ref[...]
    t = t_ref[...]
    r = r_ref[...]
    henc = h[:, 160:192] * jnp.sin(h[:, 96:128] * tvec + h[:, 128:160])
    tenc = t[:, 160:192] * jnp.sin(t[:, 96:128] * tvec + t[:, 128:160])
    denc = henc - tenc
    d10 = (denc[:, 0:10] + denc[:, 10:20] + denc[:, 20:30] + r[:, 96:106])
    d96 = h[:, :96] + r[:, :96] - t[:, :96]
    s = jnp.sum(d96 * d96, axis=1) + jnp.sum(d10 * d10, axis=1)
    out_ref[...] = jnp.sqrt(s)[:, None]


def _tc_score(h, t, r, y2, m2, d2, interpret=False):
    return pl.pallas_call(
        _tc_score_body,
        grid=(BATCH // _BLK,),
        in_specs=[
            pl.BlockSpec((_BLK, MEGA_W), lambda i: (i, 0)),
            pl.BlockSpec((_BLK, MEGA_W), lambda i: (i, 0)),
            pl.BlockSpec((_BLK, REL_W), lambda i: (i, 0)),
            pl.BlockSpec((_BLK, 1), lambda i: (i, 0)),
            pl.BlockSpec((_BLK, 1), lambda i: (i, 0)),
            pl.BlockSpec((_BLK, 1), lambda i: (i, 0)),
        ],
        out_specs=pl.BlockSpec((_BLK, 1), lambda i: (i, 0)),
        out_shape=jax.ShapeDtypeStruct((BATCH, 1), jnp.float32),
        interpret=interpret,
    )(h, t, r, y2, m2, d2)


def _pack_tables(entity_emb, relation_emb, year_freq, month_freq, day_freq,
                 year_phi, month_phi, day_phi, year_amp, month_amp, day_amp):
    zn = lambda w: jnp.zeros((NUM_ENT, w), jnp.float32)
    mega = jnp.concatenate(
        [entity_emb, zn(6),
         year_freq, month_freq, day_freq, zn(2),
         year_phi, month_phi, day_phi, zn(2),
         year_amp, month_amp, day_amp, zn(66)], axis=1)
    zr = lambda w: jnp.zeros((NUM_REL, w), jnp.float32)
    relpad = jnp.concatenate(
        [relation_emb[:, :ENT_DIM], zr(6), relation_emb[:, ENT_DIM:], zr(22)],
        axis=1)
    return mega, relpad


def kernel(heads, rels, tails, years, months, days, entity_emb, relation_emb,
           year_freq, month_freq, day_freq, year_phi, month_phi, day_phi,
           year_amp, month_amp, day_amp):
    mega, relpad = _pack_tables(entity_emb, relation_emb, year_freq,
                                month_freq, day_freq, year_phi, month_phi,
                                day_phi, year_amp, month_amp, day_amp)
    hi = heads.astype(jnp.int32)
    ti = tails.astype(jnp.int32)
    ri = rels.astype(jnp.int32)
    h, t, r = _get_sc_gather()(hi, ti, ri, mega, relpad)
    y2 = years.reshape(BATCH, 1)
    m2 = months.reshape(BATCH, 1)
    d2 = days.reshape(BATCH, 1)
    scores = _tc_score(h, t, r, y2, m2, d2)
    return scores.reshape(-1)
